# Initial kernel scaffold; baseline (speedup 1.0000x reference)
import jax
import jax.numpy as jnp
from jax.experimental import pallas as pl


def kernel(nf, W, src, dst, neg):
    raise NotImplementedError("write your pallas kernel here")


# speedup vs baseline: 2.7763x; 2.7763x over previous
import functools

import jax
import jax.numpy as jnp
from jax import lax
from jax.experimental import pallas as pl
from jax.experimental.pallas import tpu as pltpu
from jax.experimental.pallas import tpu_sc as plsc

D_MODEL = 256
NC, NS = 2, 16
NW = NC * NS
CHUNK = 64


def _matmul_body(nf_ref, w_ref, out_ref):
    out_ref[...] = jnp.dot(nf_ref[...], w_ref[...],
                           preferred_element_type=jnp.float32)


def _project(nf, W):
    n, d = nf.shape
    bm = 2000 if n % 2000 == 0 else 512
    return pl.pallas_call(
        _matmul_body,
        grid=(pl.cdiv(n, bm),),
        in_specs=[pl.BlockSpec((bm, d), lambda i: (i, 0)),
                  pl.BlockSpec((d, d), lambda i: (0, 0))],
        out_specs=pl.BlockSpec((bm, d), lambda i: (i, 0)),
        out_shape=jax.ShapeDtypeStruct((n, d), jnp.float32),
    )(nf, W)


def _sc_scores(h, src, dst, neg, ridx_t):
    b = src.shape[0]
    epw = b // NW
    nchunks = epw // CHUNK
    mesh = plsc.VectorSubcoreMesh(core_axis_name="c", subcore_axis_name="s",
                                  num_cores=NC, num_subcores=NS)

    @functools.partial(
        pl.kernel,
        out_type=jax.ShapeDtypeStruct((4, b), jnp.float32),
        mesh=mesh,
        compiler_params=pltpu.CompilerParams(needs_layout_passes=False),
        scratch_types=[
            pltpu.VMEM((CHUNK,), jnp.int32),
            pltpu.VMEM((CHUNK,), jnp.int32),
            [pltpu.VMEM((CHUNK,), jnp.int32) for _ in range(3)],
            [pltpu.VMEM((CHUNK,), jnp.int32) for _ in range(3)],
            pltpu.VMEM((CHUNK, D_MODEL), jnp.float32),
            pltpu.VMEM((CHUNK, D_MODEL), jnp.float32),
            [pltpu.VMEM((CHUNK, D_MODEL), jnp.float32) for _ in range(3)],
            [pltpu.VMEM((CHUNK,), jnp.float32) for _ in range(4)],
            [pltpu.SemaphoreType.DMA for _ in range(5)],
        ],
    )
    def sc_kernel(h_hbm, src_hbm, dst_hbm, neg_hbm, ridx_hbm, out_hbm,
                  sidx, didx, rchunk, negid, srows, drows, nrows, scores,
                  sems):
        wid = lax.axis_index("s") * NC + lax.axis_index("c")
        lane = lax.iota(jnp.int32, 16)

        def chunk_body(ci, carry):
            base = wid * epw + ci * CHUNK
            pltpu.sync_copy(src_hbm.at[pl.ds(base, CHUNK)], sidx)
            pltpu.sync_copy(dst_hbm.at[pl.ds(base, CHUNK)], didx)
            for k in range(3):
                pltpu.sync_copy(ridx_hbm.at[k, pl.ds(base, CHUNK)], rchunk[k])
            negc = [pltpu.async_copy(neg_hbm.at[rchunk[k]], negid[k], sems[4])
                    for k in range(3)]
            for c in negc:
                c.wait()
            cps = [pltpu.async_copy(h_hbm.at[sidx], srows, sems[0]),
                   pltpu.async_copy(h_hbm.at[didx], drows, sems[1])]
            cps += [pltpu.async_copy(h_hbm.at[negid[k]], nrows[k], sems[2 + (k % 2)])
                    for k in range(3)]
            for c in cps:
                c.wait()

            for g in range(CHUNK // 16):
                erow = lane + (g * 16)
                zeros = jnp.zeros((16,), jnp.float32)

                def jbody(j, accs, _erow=erow):
                    a_p, a_0, a_1, a_2 = accs
                    col = jnp.full((16,), 0, jnp.int32) + j
                    sv = plsc.load_gather(srows, [_erow, col])
                    dv = plsc.load_gather(drows, [_erow, col])
                    n0 = plsc.load_gather(nrows[0], [_erow, col])
                    n1 = plsc.load_gather(nrows[1], [_erow, col])
                    n2 = plsc.load_gather(nrows[2], [_erow, col])
                    return (a_p + sv * dv, a_0 + sv * n0,
                            a_1 + sv * n1, a_2 + sv * n2)

                a_p, a_0, a_1, a_2 = lax.fori_loop(
                    0, D_MODEL, jbody, (zeros, zeros, zeros, zeros))
                scores[0][pl.ds(g * 16, 16)] = a_p
                scores[1][pl.ds(g * 16, 16)] = a_0
                scores[2][pl.ds(g * 16, 16)] = a_1
                scores[3][pl.ds(g * 16, 16)] = a_2

            for k in range(4):
                pltpu.sync_copy(scores[k], out_hbm.at[k, pl.ds(base, CHUNK)])
            return carry

        lax.fori_loop(0, nchunks, chunk_body, 0)

    return sc_kernel(h, src, dst, neg, ridx_t)


def _loss_body(s_ref, out_ref):
    x = s_ref[...]
    sp = jnp.maximum(x, 0.0) + jnp.log1p(jnp.exp(-jnp.abs(x)))
    spm = sp - x
    out_ref[...] = spm[0] + 10.0 * (sp[1] + sp[2] + sp[3])


def _loss(scores):
    b = scores.shape[1]
    nb = 8192
    return pl.pallas_call(
        _loss_body,
        grid=(b // nb,),
        in_specs=[pl.BlockSpec((4, nb), lambda i: (0, i))],
        out_specs=pl.BlockSpec((nb,), lambda i: (i,)),
        out_shape=jax.ShapeDtypeStruct((b,), jnp.float32),
    )(scores)


def kernel(nf, W, src, dst, neg):
    b = src.shape[0]
    h = _project(nf, W)
    ridx = jax.random.randint(jax.random.key(42), (b, 3), 0, b)
    ridx_t = ridx.T.astype(jnp.int32)
    scores = _sc_scores(h, src.astype(jnp.int32), dst.astype(jnp.int32),
                        neg.astype(jnp.int32), ridx_t)
    return _loss(scores)


# speedup vs baseline: 8.5523x; 3.0804x over previous
import functools

import jax
import jax.numpy as jnp
from jax import lax
from jax.experimental import pallas as pl
from jax.experimental.pallas import tpu as pltpu
from jax.experimental.pallas import tpu_sc as plsc

D_MODEL = 256
NC, NS = 2, 16
NW = NC * NS
CHUNK = 64


def _matmul_body(nf_ref, w_ref, out_ref):
    out_ref[...] = jnp.dot(nf_ref[...], w_ref[...],
                           preferred_element_type=jnp.float32)


def _project(nf, W):
    n, d = nf.shape
    bm = 2000 if n % 2000 == 0 else 512
    return pl.pallas_call(
        _matmul_body,
        grid=(pl.cdiv(n, bm),),
        in_specs=[pl.BlockSpec((bm, d), lambda i: (i, 0)),
                  pl.BlockSpec((d, d), lambda i: (0, 0))],
        out_specs=pl.BlockSpec((bm, d), lambda i: (i, 0)),
        out_shape=jax.ShapeDtypeStruct((n, d), jnp.float32),
    )(nf, W)


def _sc_gather(h, src, dst, neg, ridx_t):
    b = src.shape[0]
    epw = b // NW
    nchunks = epw // CHUNK
    mesh = plsc.VectorSubcoreMesh(core_axis_name="c", subcore_axis_name="s",
                                  num_cores=NC, num_subcores=NS)

    @functools.partial(
        pl.kernel,
        out_type=jax.ShapeDtypeStruct((5, b, D_MODEL), jnp.float32),
        mesh=mesh,
        compiler_params=pltpu.CompilerParams(needs_layout_passes=False),
        scratch_types=[
            pltpu.VMEM((CHUNK,), jnp.int32),
            pltpu.VMEM((CHUNK,), jnp.int32),
            [pltpu.VMEM((CHUNK,), jnp.int32) for _ in range(3)],
            [pltpu.VMEM((CHUNK,), jnp.int32) for _ in range(3)],
            pltpu.VMEM((CHUNK, D_MODEL), jnp.float32),
            pltpu.VMEM((CHUNK, D_MODEL), jnp.float32),
            [pltpu.VMEM((CHUNK, D_MODEL), jnp.float32) for _ in range(3)],
            [pltpu.SemaphoreType.DMA for _ in range(6)],
        ],
    )
    def sc_kernel(h_hbm, src_hbm, dst_hbm, neg_hbm, ridx_hbm, out_hbm,
                  sidx, didx, rchunk, negid, srows, drows, nrows, sems):
        wid = lax.axis_index("s") * NC + lax.axis_index("c")

        def chunk_body(ci, carry):
            base = wid * epw + ci * CHUNK
            pltpu.sync_copy(src_hbm.at[pl.ds(base, CHUNK)], sidx)
            pltpu.sync_copy(dst_hbm.at[pl.ds(base, CHUNK)], didx)
            for k in range(3):
                pltpu.sync_copy(ridx_hbm.at[k, pl.ds(base, CHUNK)], rchunk[k])
            negc = [pltpu.async_copy(neg_hbm.at[rchunk[k]], negid[k], sems[5])
                    for k in range(3)]
            for c in negc:
                c.wait()
            cps = [pltpu.async_copy(h_hbm.at[sidx], srows, sems[0]),
                   pltpu.async_copy(h_hbm.at[didx], drows, sems[1])]
            cps += [pltpu.async_copy(h_hbm.at[negid[k]], nrows[k], sems[2 + k])
                    for k in range(3)]
            outs = []
            for k, rows in enumerate([srows, drows] + nrows):
                cps[k].wait()
                outs.append(pltpu.async_copy(
                    rows, out_hbm.at[k, pl.ds(base, CHUNK)], sems[k]))
            for c in outs:
                c.wait()
            return carry

        lax.fori_loop(0, nchunks, chunk_body, 0)

    return sc_kernel(h, src, dst, neg, ridx_t)


def _dot_loss_body(g_ref, out_ref):
    g = g_ref[...]
    s = g[0]
    pos = jnp.sum(s * g[1], axis=-1)
    n0 = jnp.sum(s * g[2], axis=-1)
    n1 = jnp.sum(s * g[3], axis=-1)
    n2 = jnp.sum(s * g[4], axis=-1)

    def sp(x):
        return jnp.maximum(x, 0.0) + jnp.log1p(jnp.exp(-jnp.abs(x)))

    out_ref[...] = (sp(-pos)) + 10.0 * (sp(n0) + sp(n1) + sp(n2))


def _dot_loss(g):
    b = g.shape[1]
    nb = 2048
    return pl.pallas_call(
        _dot_loss_body,
        grid=(b // nb,),
        in_specs=[pl.BlockSpec((5, nb, D_MODEL), lambda i: (0, i, 0))],
        out_specs=pl.BlockSpec((nb,), lambda i: (i,)),
        out_shape=jax.ShapeDtypeStruct((b,), jnp.float32),
    )(g)


def kernel(nf, W, src, dst, neg):
    b = src.shape[0]
    h = _project(nf, W)
    ridx = jax.random.randint(jax.random.key(42), (b, 3), 0, b)
    ridx_t = ridx.T.astype(jnp.int32)
    g = _sc_gather(h, src.astype(jnp.int32), dst.astype(jnp.int32),
                   neg.astype(jnp.int32), ridx_t)
    return _dot_loss(g)


# speedup vs baseline: 10.6659x; 1.2471x over previous
import functools

import jax
import jax.numpy as jnp
from jax import lax
from jax.experimental import pallas as pl
from jax.experimental.pallas import tpu as pltpu
from jax.experimental.pallas import tpu_sc as plsc

D_MODEL = 256
NC, NS = 2, 16
NW = NC * NS
CHUNK = 32


def _matmul_body(nf_ref, w_ref, out_ref):
    out_ref[...] = jnp.dot(nf_ref[...], w_ref[...],
                           preferred_element_type=jnp.float32)


def _project(nf, W):
    n, d = nf.shape
    bm = 2000 if n % 2000 == 0 else 512
    return pl.pallas_call(
        _matmul_body,
        grid=(pl.cdiv(n, bm),),
        in_specs=[pl.BlockSpec((bm, d), lambda i: (i, 0)),
                  pl.BlockSpec((d, d), lambda i: (0, 0))],
        out_specs=pl.BlockSpec((bm, d), lambda i: (i, 0)),
        out_shape=jax.ShapeDtypeStruct((n, d), jnp.float32),
    )(nf, W)


def _sc_gather(h, src, dst, neg, r0, r1, r2):
    b = src.shape[0]
    epw = b // NW
    nchunks = epw // CHUNK
    nbuf = 2
    mesh = plsc.VectorSubcoreMesh(core_axis_name="c", subcore_axis_name="s",
                                  num_cores=NC, num_subcores=NS)

    @functools.partial(
        pl.kernel,
        out_type=jax.ShapeDtypeStruct((5, b, D_MODEL), jnp.float32),
        mesh=mesh,
        compiler_params=pltpu.CompilerParams(needs_layout_passes=False),
        scratch_types=[
            pltpu.VMEM((epw,), jnp.int32),
            pltpu.VMEM((epw,), jnp.int32),
            [pltpu.VMEM((epw,), jnp.int32) for _ in range(3)],
            [pltpu.VMEM((epw,), jnp.int32) for _ in range(3)],
            [[pltpu.VMEM((CHUNK, D_MODEL), jnp.float32) for _ in range(5)]
             for _ in range(nbuf)],
            [pltpu.SemaphoreType.DMA for _ in range(2 * nbuf + 1)],
        ],
    )
    def sc_kernel(h_hbm, src_hbm, dst_hbm, neg_hbm, r0_hbm, r1_hbm, r2_hbm,
                  out_hbm, sidx, didx, rall, negid, rows, sems):
        ridx_hbm = [r0_hbm, r1_hbm, r2_hbm]
        wid = lax.axis_index("s") * NC + lax.axis_index("c")
        wbase = wid * epw
        gsem = sems[:nbuf]
        wsem = sems[nbuf:2 * nbuf]

        pltpu.sync_copy(src_hbm.at[pl.ds(wbase, epw)], sidx)
        pltpu.sync_copy(dst_hbm.at[pl.ds(wbase, epw)], didx)
        for k in range(3):
            pltpu.sync_copy(ridx_hbm[k].at[pl.ds(wbase, epw)], rall[k])
        negc = [pltpu.async_copy(neg_hbm.at[rall[k]], negid[k], sems[-1])
                for k in range(3)]
        for c in negc:
            c.wait()

        idx5 = [sidx, didx] + negid

        def idx_slice(k, off):
            return idx5[k].at[pl.ds(off, CHUNK)]

        def g_issue(ci, bslot):
            off = ci * CHUNK
            for k in range(5):
                pltpu.async_copy(h_hbm.at[idx_slice(k, off)],
                                 rows[bslot][k], gsem[bslot])

        def g_drain(bslot):
            for k in range(5):
                pltpu.make_async_copy(h_hbm.at[pl.ds(0, CHUNK)],
                                      rows[bslot][k], gsem[bslot]).wait()

        for bslot in range(nbuf):
            g_issue(bslot, bslot)

        def pair_body(ci, carry):
            for bslot in range(nbuf):
                chunk = ci + bslot
                base = wbase + chunk * CHUNK
                g_drain(bslot)
                wcp = [pltpu.async_copy(rows[bslot][k],
                                        out_hbm.at[k, pl.ds(base, CHUNK)],
                                        wsem[bslot])
                       for k in range(5)]
                for c in wcp:
                    c.wait()
                g_issue(lax.rem(chunk + nbuf, nchunks), bslot)
            return carry

        lax.fori_loop(0, nchunks // nbuf, lambda i, c: pair_body(i * nbuf, c),
                      0)
        for bslot in range(nbuf):
            g_drain(bslot)

    return sc_kernel(h, src, dst, neg, r0, r1, r2)


def _dot_loss_body(g_ref, out_ref):
    g = g_ref[...]
    s = g[0]
    pos = jnp.sum(s * g[1], axis=-1)
    n0 = jnp.sum(s * g[2], axis=-1)
    n1 = jnp.sum(s * g[3], axis=-1)
    n2 = jnp.sum(s * g[4], axis=-1)

    def sp(x):
        return jnp.maximum(x, 0.0) + jnp.log1p(jnp.exp(-jnp.abs(x)))

    out_ref[...] = (sp(-pos)) + 10.0 * (sp(n0) + sp(n1) + sp(n2))


def _dot_loss(g):
    b = g.shape[1]
    nb = 2048
    return pl.pallas_call(
        _dot_loss_body,
        grid=(b // nb,),
        in_specs=[pl.BlockSpec((5, nb, D_MODEL), lambda i: (0, i, 0))],
        out_specs=pl.BlockSpec((nb,), lambda i: (i,)),
        out_shape=jax.ShapeDtypeStruct((b,), jnp.float32),
    )(g)


def kernel(nf, W, src, dst, neg):
    b = src.shape[0]
    h = _project(nf, W)
    ridx = jax.random.randint(jax.random.key(42), (b, 3), 0, b)
    r0, r1, r2 = (ridx[:, k].astype(jnp.int32) for k in range(3))
    g = _sc_gather(h, src.astype(jnp.int32), dst.astype(jnp.int32),
                   neg.astype(jnp.int32), r0, r1, r2)
    return _dot_loss(g)


# speedup vs baseline: 14.8040x; 1.3880x over previous
import functools

import jax
import jax.numpy as jnp
from jax import lax
from jax.experimental import pallas as pl
from jax.experimental.pallas import tpu as pltpu
from jax.experimental.pallas import tpu_sc as plsc

D_MODEL = 256
D_WORDS = D_MODEL // 2
NC, NS = 2, 16
NW = NC * NS
CHUNK = 32


def _rne_bf16_hi(x):
    u = lax.bitcast_convert_type(x, jnp.uint32)
    r = u + jnp.uint32(0x7FFF) + ((u >> jnp.uint32(16)) & jnp.uint32(1))
    return r & jnp.uint32(0xFFFF0000)


def _matmul_body(nf_ref, w_ref, out_ref):
    acc = jnp.dot(nf_ref[...], w_ref[...],
                  preferred_element_type=jnp.float32)
    lo = _rne_bf16_hi(acc[:, :D_WORDS]) >> jnp.uint32(16)
    hi = _rne_bf16_hi(acc[:, D_WORDS:])
    out_ref[...] = lax.bitcast_convert_type(hi | lo, jnp.int32)


def _project(nf, W):
    n, d = nf.shape
    bm = 2000 if n % 2000 == 0 else 512
    return pl.pallas_call(
        _matmul_body,
        grid=(pl.cdiv(n, bm),),
        in_specs=[pl.BlockSpec((bm, d), lambda i: (i, 0)),
                  pl.BlockSpec((d, d), lambda i: (0, 0))],
        out_specs=pl.BlockSpec((bm, D_WORDS), lambda i: (i, 0)),
        out_shape=jax.ShapeDtypeStruct((n, D_WORDS), jnp.int32),
    )(nf, W)


def _sc_gather(h, src, dst, neg, r0, r1, r2):
    b = src.shape[0]
    epw = b // NW
    nchunks = epw // CHUNK
    nbuf = 2
    mesh = plsc.VectorSubcoreMesh(core_axis_name="c", subcore_axis_name="s",
                                  num_cores=NC, num_subcores=NS)

    @functools.partial(
        pl.kernel,
        out_type=jax.ShapeDtypeStruct((5, b, D_WORDS), jnp.int32),
        mesh=mesh,
        compiler_params=pltpu.CompilerParams(needs_layout_passes=False),
        scratch_types=[
            pltpu.VMEM((epw,), jnp.int32),
            pltpu.VMEM((epw,), jnp.int32),
            [pltpu.VMEM((epw,), jnp.int32) for _ in range(3)],
            [pltpu.VMEM((epw,), jnp.int32) for _ in range(3)],
            [[pltpu.VMEM((CHUNK, D_WORDS), jnp.int32) for _ in range(5)]
             for _ in range(nbuf)],
            [pltpu.SemaphoreType.DMA for _ in range(2 * nbuf + 1)],
        ],
    )
    def sc_kernel(h_hbm, src_hbm, dst_hbm, neg_hbm, r0_hbm, r1_hbm, r2_hbm,
                  out_hbm, sidx, didx, rall, negid, rows, sems):
        ridx_hbm = [r0_hbm, r1_hbm, r2_hbm]
        wid = lax.axis_index("s") * NC + lax.axis_index("c")
        wbase = wid * epw
        gsem = sems[:nbuf]
        wsem = sems[nbuf:2 * nbuf]

        pltpu.sync_copy(src_hbm.at[pl.ds(wbase, epw)], sidx)
        pltpu.sync_copy(dst_hbm.at[pl.ds(wbase, epw)], didx)
        for k in range(3):
            pltpu.sync_copy(ridx_hbm[k].at[pl.ds(wbase, epw)], rall[k])
        negc = [pltpu.async_copy(neg_hbm.at[rall[k]], negid[k], sems[-1])
                for k in range(3)]
        for c in negc:
            c.wait()

        idx5 = [sidx, didx] + negid

        def idx_slice(k, off):
            return idx5[k].at[pl.ds(off, CHUNK)]

        def g_issue(ci, bslot):
            off = ci * CHUNK
            for k in range(5):
                pltpu.async_copy(h_hbm.at[idx_slice(k, off)],
                                 rows[bslot][k], gsem[bslot])

        def g_drain(bslot):
            for k in range(5):
                pltpu.make_async_copy(h_hbm.at[pl.ds(0, CHUNK)],
                                      rows[bslot][k], gsem[bslot]).wait()

        for bslot in range(nbuf):
            g_issue(bslot, bslot)

        def pair_body(ci, carry):
            for bslot in range(nbuf):
                chunk = ci + bslot
                base = wbase + chunk * CHUNK
                g_drain(bslot)
                wcp = [pltpu.async_copy(rows[bslot][k],
                                        out_hbm.at[k, pl.ds(base, CHUNK)],
                                        wsem[bslot])
                       for k in range(5)]
                for c in wcp:
                    c.wait()
                g_issue(lax.rem(chunk + nbuf, nchunks), bslot)
            return carry

        lax.fori_loop(0, nchunks // nbuf, lambda i, c: pair_body(i * nbuf, c),
                      0)
        for bslot in range(nbuf):
            g_drain(bslot)

    return sc_kernel(h, src, dst, neg, r0, r1, r2)


def _dot_loss_body(g_ref, out_ref):
    u = lax.bitcast_convert_type(g_ref[...], jnp.uint32)
    flo = lax.bitcast_convert_type(u << jnp.uint32(16), jnp.float32)
    fhi = lax.bitcast_convert_type(u & jnp.uint32(0xFFFF0000), jnp.float32)

    def dot(k):
        return jnp.sum(flo[0] * flo[k] + fhi[0] * fhi[k], axis=-1)

    pos, n0, n1, n2 = dot(1), dot(2), dot(3), dot(4)

    def sp(x):
        return jnp.maximum(x, 0.0) + jnp.log1p(jnp.exp(-jnp.abs(x)))

    out_ref[...] = (sp(-pos)) + 10.0 * (sp(n0) + sp(n1) + sp(n2))


def _dot_loss(g):
    b = g.shape[1]
    nb = 2048
    return pl.pallas_call(
        _dot_loss_body,
        grid=(b // nb,),
        in_specs=[pl.BlockSpec((5, nb, D_WORDS), lambda i: (0, i, 0))],
        out_specs=pl.BlockSpec((nb,), lambda i: (i,)),
        out_shape=jax.ShapeDtypeStruct((b,), jnp.float32),
    )(g)


def kernel(nf, W, src, dst, neg):
    b = src.shape[0]
    h = _project(nf, W)
    ridx = jax.random.randint(jax.random.key(42), (b, 3), 0, b)
    r0, r1, r2 = (ridx[:, k].astype(jnp.int32) for k in range(3))
    g = _sc_gather(h, src.astype(jnp.int32), dst.astype(jnp.int32),
                   neg.astype(jnp.int32), r0, r1, r2)
    return _dot_loss(g)


# speedup vs baseline: 15.9348x; 1.0764x over previous
import functools

import jax
import jax.numpy as jnp
from jax import lax
from jax.experimental import pallas as pl
from jax.experimental.pallas import tpu as pltpu
from jax.experimental.pallas import tpu_sc as plsc

D_MODEL = 256
D_WORDS = D_MODEL // 2
NC, NS = 2, 16
NW = NC * NS
CHUNK = 32


def _rne_bf16_hi(x):
    u = lax.bitcast_convert_type(x, jnp.uint32)
    r = u + jnp.uint32(0x7FFF) + ((u >> jnp.uint32(16)) & jnp.uint32(1))
    return r & jnp.uint32(0xFFFF0000)


def _matmul_body(nf_ref, w_ref, out_ref):
    acc = jnp.dot(nf_ref[...], w_ref[...],
                  preferred_element_type=jnp.float32)
    lo = _rne_bf16_hi(acc[:, :D_WORDS]) >> jnp.uint32(16)
    hi = _rne_bf16_hi(acc[:, D_WORDS:])
    out_ref[...] = lax.bitcast_convert_type(hi | lo, jnp.int32)


def _project(nf, W):
    n, d = nf.shape
    bm = 2000 if n % 2000 == 0 else 512
    return pl.pallas_call(
        _matmul_body,
        grid=(pl.cdiv(n, bm),),
        in_specs=[pl.BlockSpec((bm, d), lambda i: (i, 0)),
                  pl.BlockSpec((d, d), lambda i: (0, 0))],
        out_specs=pl.BlockSpec((bm, D_WORDS), lambda i: (i, 0)),
        out_shape=jax.ShapeDtypeStruct((n, D_WORDS), jnp.int32),
    )(nf, W)


def _sc_gather(h, src, dst, neg, r0, r1, r2):
    b = src.shape[0]
    epw = b // NW
    nchunks = epw // CHUNK
    nbuf = 2
    mesh = plsc.VectorSubcoreMesh(core_axis_name="c", subcore_axis_name="s",
                                  num_cores=NC, num_subcores=NS)

    @functools.partial(
        pl.kernel,
        out_type=jax.ShapeDtypeStruct((5, b, D_WORDS), jnp.int32),
        mesh=mesh,
        compiler_params=pltpu.CompilerParams(needs_layout_passes=False),
        scratch_types=[
            pltpu.VMEM((epw,), jnp.int32),
            pltpu.VMEM((epw,), jnp.int32),
            [pltpu.VMEM((epw,), jnp.int32) for _ in range(3)],
            [pltpu.VMEM((epw,), jnp.int32) for _ in range(3)],
            [[pltpu.VMEM((CHUNK, D_WORDS), jnp.int32) for _ in range(5)]
             for _ in range(nbuf)],
            [pltpu.SemaphoreType.DMA for _ in range(2 * nbuf + 1)],
        ],
    )
    def sc_kernel(h_hbm, src_hbm, dst_hbm, neg_hbm, r0_hbm, r1_hbm, r2_hbm,
                  out_hbm, sidx, didx, rall, negid, rows, sems):
        ridx_hbm = [r0_hbm, r1_hbm, r2_hbm]
        wid = lax.axis_index("s") * NC + lax.axis_index("c")
        wbase = wid * epw
        gsem = sems[:nbuf]
        wsem = sems[nbuf:2 * nbuf]

        pltpu.sync_copy(src_hbm.at[pl.ds(wbase, epw)], sidx)
        pltpu.sync_copy(dst_hbm.at[pl.ds(wbase, epw)], didx)
        for k in range(3):
            pltpu.sync_copy(ridx_hbm[k].at[pl.ds(wbase, epw)], rall[k])
        negc = [pltpu.async_copy(neg_hbm.at[rall[k]], negid[k], sems[-1])
                for k in range(3)]
        for c in negc:
            c.wait()

        idx5 = [sidx, didx] + negid

        def idx_slice(k, off):
            return idx5[k].at[pl.ds(off, CHUNK)]

        def g_issue(ci, bslot):
            off = ci * CHUNK
            for k in range(5):
                pltpu.async_copy(h_hbm.at[idx_slice(k, off)],
                                 rows[bslot][k], gsem[bslot])

        def g_drain(bslot):
            for k in range(5):
                pltpu.make_async_copy(h_hbm.at[pl.ds(0, CHUNK)],
                                      rows[bslot][k], gsem[bslot]).wait()

        for bslot in range(nbuf):
            g_issue(bslot, bslot)

        def pair_body(ci, carry):
            for bslot in range(nbuf):
                chunk = ci + bslot
                base = wbase + chunk * CHUNK
                g_drain(bslot)
                wcp = [pltpu.async_copy(rows[bslot][k],
                                        out_hbm.at[k, pl.ds(base, CHUNK)],
                                        wsem[bslot])
                       for k in range(5)]
                for c in wcp:
                    c.wait()
                g_issue(lax.rem(chunk + nbuf, nchunks), bslot)
            return carry

        lax.fori_loop(0, nchunks // nbuf, lambda i, c: pair_body(i * nbuf, c),
                      0)
        for bslot in range(nbuf):
            g_drain(bslot)

    return sc_kernel(h, src, dst, neg, r0, r1, r2)


def _dot_loss_body(g_ref, out_ref):
    u = lax.bitcast_convert_type(g_ref[...], jnp.uint32)
    flo = lax.bitcast_convert_type(u << jnp.uint32(16), jnp.float32)
    fhi = lax.bitcast_convert_type(u & jnp.uint32(0xFFFF0000), jnp.float32)

    def dot(k):
        return jnp.sum(flo[0] * flo[k] + fhi[0] * fhi[k], axis=-1)

    pos, n0, n1, n2 = dot(1), dot(2), dot(3), dot(4)

    def sp(x):
        return jnp.maximum(x, 0.0) + jnp.log1p(jnp.exp(-jnp.abs(x)))

    out_ref[...] = (sp(-pos)) + 10.0 * (sp(n0) + sp(n1) + sp(n2))


def _dot_loss(g):
    b = g.shape[1]
    nb = 2048
    return pl.pallas_call(
        _dot_loss_body,
        grid=(b // nb,),
        in_specs=[pl.BlockSpec((5, nb, D_WORDS), lambda i: (0, i, 0))],
        out_specs=pl.BlockSpec((nb,), lambda i: (i,)),
        out_shape=jax.ShapeDtypeStruct((b,), jnp.float32),
    )(g)


def kernel(nf, W, src, dst, neg):
    b = src.shape[0]
    h = _project(nf, W)
    ridx = jax.random.randint(jax.random.key(42), (b, 3), 0, b)
    r0, r1, r2 = (ridx[:, k].astype(jnp.int32) for k in range(3))
    src32, dst32 = src.astype(jnp.int32), dst.astype(jnp.int32)
    neg32 = neg.astype(jnp.int32)
    nch = 2 if b % (2 * NW * CHUNK * 2) == 0 else 1
    cb = b // nch
    outs = []
    for i in range(nch):
        lo, hi = i * cb, (i + 1) * cb
        g = _sc_gather(h, src32[lo:hi], dst32[lo:hi], neg32,
                       r0[lo:hi], r1[lo:hi], r2[lo:hi])
        outs.append(_dot_loss(g))
    return jnp.concatenate(outs) if nch > 1 else outs[0]


# speedup vs baseline: 16.0114x; 1.0048x over previous
import functools

import jax
import jax.numpy as jnp
from jax import lax
from jax.experimental import pallas as pl
from jax.experimental.pallas import tpu as pltpu
from jax.experimental.pallas import tpu_sc as plsc

D_MODEL = 256
D_WORDS = D_MODEL // 2
NC, NS = 2, 16
NW = NC * NS
CHUNK = 32


def _rne_bf16_hi(x):
    u = lax.bitcast_convert_type(x, jnp.uint32)
    r = u + jnp.uint32(0x7FFF) + ((u >> jnp.uint32(16)) & jnp.uint32(1))
    return r & jnp.uint32(0xFFFF0000)


def _matmul_body(nf_ref, w_ref, out_ref):
    acc = jnp.dot(nf_ref[...], w_ref[...],
                  preferred_element_type=jnp.float32)
    lo = _rne_bf16_hi(acc[:, :D_WORDS]) >> jnp.uint32(16)
    hi = _rne_bf16_hi(acc[:, D_WORDS:])
    out_ref[...] = lax.bitcast_convert_type(hi | lo, jnp.int32)


def _project(nf, W):
    n, d = nf.shape
    bm = 2000 if n % 2000 == 0 else 512
    return pl.pallas_call(
        _matmul_body,
        grid=(pl.cdiv(n, bm),),
        in_specs=[pl.BlockSpec((bm, d), lambda i: (i, 0)),
                  pl.BlockSpec((d, d), lambda i: (0, 0))],
        out_specs=pl.BlockSpec((bm, D_WORDS), lambda i: (i, 0)),
        out_shape=jax.ShapeDtypeStruct((n, D_WORDS), jnp.int32),
    )(nf, W)


def _sc_gather(h, src, dst, neg, r0, r1, r2):
    b = src.shape[0]
    epw = b // NW
    nchunks = epw // CHUNK
    nbuf = 2
    mesh = plsc.VectorSubcoreMesh(core_axis_name="c", subcore_axis_name="s",
                                  num_cores=NC, num_subcores=NS)

    @functools.partial(
        pl.kernel,
        out_type=jax.ShapeDtypeStruct((5, b, D_WORDS), jnp.int32),
        mesh=mesh,
        compiler_params=pltpu.CompilerParams(needs_layout_passes=False),
        scratch_types=[
            pltpu.VMEM((epw,), jnp.int32),
            pltpu.VMEM((epw,), jnp.int32),
            [pltpu.VMEM((epw,), jnp.int32) for _ in range(3)],
            [pltpu.VMEM((epw,), jnp.int32) for _ in range(3)],
            [[pltpu.VMEM((CHUNK, D_WORDS), jnp.int32) for _ in range(5)]
             for _ in range(nbuf)],
            [pltpu.SemaphoreType.DMA for _ in range(2 * nbuf + 1)],
        ],
    )
    def sc_kernel(h_hbm, src_hbm, dst_hbm, neg_hbm, r0_hbm, r1_hbm, r2_hbm,
                  out_hbm, sidx, didx, rall, negid, rows, sems):
        ridx_hbm = [r0_hbm, r1_hbm, r2_hbm]
        wid = lax.axis_index("s") * NC + lax.axis_index("c")
        wbase = wid * epw
        gsem = sems[:nbuf]
        wsem = sems[nbuf:2 * nbuf]

        pltpu.sync_copy(src_hbm.at[pl.ds(wbase, epw)], sidx)
        pltpu.sync_copy(dst_hbm.at[pl.ds(wbase, epw)], didx)
        for k in range(3):
            pltpu.sync_copy(ridx_hbm[k].at[pl.ds(wbase, epw)], rall[k])
        negc = [pltpu.async_copy(neg_hbm.at[rall[k]], negid[k], sems[-1])
                for k in range(3)]
        for c in negc:
            c.wait()

        idx5 = [sidx, didx] + negid

        def idx_slice(k, off):
            return idx5[k].at[pl.ds(off, CHUNK)]

        def g_issue(ci, bslot):
            off = ci * CHUNK
            for k in range(5):
                pltpu.async_copy(h_hbm.at[idx_slice(k, off)],
                                 rows[bslot][k], gsem[bslot])

        def g_drain(bslot):
            for k in range(5):
                pltpu.make_async_copy(h_hbm.at[pl.ds(0, CHUNK)],
                                      rows[bslot][k], gsem[bslot]).wait()

        for bslot in range(nbuf):
            g_issue(bslot, bslot)

        def pair_body(ci, carry):
            for bslot in range(nbuf):
                chunk = ci + bslot
                base = wbase + chunk * CHUNK
                g_drain(bslot)
                wcp = [pltpu.async_copy(rows[bslot][k],
                                        out_hbm.at[k, pl.ds(base, CHUNK)],
                                        wsem[bslot])
                       for k in range(5)]
                for c in wcp:
                    c.wait()
                g_issue(lax.rem(chunk + nbuf, nchunks), bslot)
            return carry

        lax.fori_loop(0, nchunks // nbuf, lambda i, c: pair_body(i * nbuf, c),
                      0)
        for bslot in range(nbuf):
            g_drain(bslot)

    return sc_kernel(h, src, dst, neg, r0, r1, r2)


def _dot_loss_body(g_ref, out_ref):
    u = lax.bitcast_convert_type(g_ref[...], jnp.uint32)
    flo = lax.bitcast_convert_type(u << jnp.uint32(16), jnp.float32)
    fhi = lax.bitcast_convert_type(u & jnp.uint32(0xFFFF0000), jnp.float32)

    def dot(k):
        return jnp.sum(flo[0] * flo[k] + fhi[0] * fhi[k], axis=-1)

    pos, n0, n1, n2 = dot(1), dot(2), dot(3), dot(4)

    def sp(x):
        return jnp.maximum(x, 0.0) + jnp.log1p(jnp.exp(-jnp.abs(x)))

    out_ref[...] = (sp(-pos)) + 10.0 * (sp(n0) + sp(n1) + sp(n2))


def _dot_loss(g):
    b = g.shape[1]
    nb = 2048
    return pl.pallas_call(
        _dot_loss_body,
        grid=(b // nb,),
        in_specs=[pl.BlockSpec((5, nb, D_WORDS), lambda i: (0, i, 0))],
        out_specs=pl.BlockSpec((nb,), lambda i: (i,)),
        out_shape=jax.ShapeDtypeStruct((b,), jnp.float32),
    )(g)


def kernel(nf, W, src, dst, neg):
    b = src.shape[0]
    h = _project(nf, W)
    ridx = jax.random.randint(jax.random.key(42), (b, 3), 0, b)
    r0, r1, r2 = (ridx[:, k].astype(jnp.int32) for k in range(3))
    src32, dst32 = src.astype(jnp.int32), dst.astype(jnp.int32)
    neg32 = neg.astype(jnp.int32)
    nch = 4 if b % (4 * NW * CHUNK * 2) == 0 else 1
    cb = b // nch
    outs = []
    for i in range(nch):
        lo, hi = i * cb, (i + 1) * cb
        g = _sc_gather(h, src32[lo:hi], dst32[lo:hi], neg32,
                       r0[lo:hi], r1[lo:hi], r2[lo:hi])
        outs.append(_dot_loss(g))
    return jnp.concatenate(outs) if nch > 1 else outs[0]


# speedup vs baseline: 16.5382x; 1.0329x over previous
import functools

import jax
import jax.numpy as jnp
from jax import lax
from jax.experimental import pallas as pl
from jax.experimental.pallas import tpu as pltpu
from jax.experimental.pallas import tpu_sc as plsc

D_MODEL = 256
D_WORDS = D_MODEL // 2
NC, NS = 2, 16
NW = NC * NS
CHUNK = 32


def _rne_bf16_hi(x):
    u = lax.bitcast_convert_type(x, jnp.uint32)
    r = u + jnp.uint32(0x7FFF) + ((u >> jnp.uint32(16)) & jnp.uint32(1))
    return r & jnp.uint32(0xFFFF0000)


def _matmul_body(nf_ref, w_ref, out_ref):
    acc = jnp.dot(nf_ref[...], w_ref[...],
                  preferred_element_type=jnp.float32)
    lo = _rne_bf16_hi(acc[:, :D_WORDS]) >> jnp.uint32(16)
    hi = _rne_bf16_hi(acc[:, D_WORDS:])
    out_ref[...] = lax.bitcast_convert_type(hi | lo, jnp.int32)


def _project(nf, W):
    n, d = nf.shape
    bm = 2000 if n % 2000 == 0 else 512
    return pl.pallas_call(
        _matmul_body,
        grid=(pl.cdiv(n, bm),),
        in_specs=[pl.BlockSpec((bm, d), lambda i: (i, 0)),
                  pl.BlockSpec((d, d), lambda i: (0, 0))],
        out_specs=pl.BlockSpec((bm, D_WORDS), lambda i: (i, 0)),
        out_shape=jax.ShapeDtypeStruct((n, D_WORDS), jnp.int32),
    )(nf, W)


def _sc_negids(neg, r0, r1, r2):
    b = r0.shape[0]
    epw = b // NW
    mesh = plsc.VectorSubcoreMesh(core_axis_name="c", subcore_axis_name="s",
                                  num_cores=NC, num_subcores=NS)

    @functools.partial(
        pl.kernel,
        out_type=[jax.ShapeDtypeStruct((b,), jnp.int32) for _ in range(3)],
        mesh=mesh,
        compiler_params=pltpu.CompilerParams(needs_layout_passes=False),
        scratch_types=[
            [pltpu.VMEM((epw,), jnp.int32) for _ in range(3)],
            [pltpu.VMEM((epw,), jnp.int32) for _ in range(3)],
            pltpu.SemaphoreType.DMA,
        ],
    )
    def negid_kernel(neg_hbm, r0_hbm, r1_hbm, r2_hbm,
                     o0_hbm, o1_hbm, o2_hbm, rall, negid, sem):
        ridx_hbm = [r0_hbm, r1_hbm, r2_hbm]
        out_hbm = [o0_hbm, o1_hbm, o2_hbm]
        wid = lax.axis_index("s") * NC + lax.axis_index("c")
        wbase = wid * epw
        for k in range(3):
            pltpu.sync_copy(ridx_hbm[k].at[pl.ds(wbase, epw)], rall[k])
        gcp = [pltpu.async_copy(neg_hbm.at[rall[k]], negid[k], sem)
               for k in range(3)]
        for c in gcp:
            c.wait()
        wcp = [pltpu.async_copy(negid[k], out_hbm[k].at[pl.ds(wbase, epw)],
                                sem)
               for k in range(3)]
        for c in wcp:
            c.wait()

    return negid_kernel(neg, r0, r1, r2)


def _sc_gather(h, src, dst, n0, n1, n2):
    b = src.shape[0]
    epw = b // NW
    nchunks = epw // CHUNK
    nbuf = 2
    mesh = plsc.VectorSubcoreMesh(core_axis_name="c", subcore_axis_name="s",
                                  num_cores=NC, num_subcores=NS)

    @functools.partial(
        pl.kernel,
        out_type=jax.ShapeDtypeStruct((5, b, D_WORDS), jnp.int32),
        mesh=mesh,
        compiler_params=pltpu.CompilerParams(needs_layout_passes=False),
        scratch_types=[
            [pltpu.VMEM((epw,), jnp.int32) for _ in range(5)],
            [[pltpu.VMEM((CHUNK, D_WORDS), jnp.int32) for _ in range(5)]
             for _ in range(nbuf)],
            [pltpu.SemaphoreType.DMA for _ in range(2 * nbuf + 1)],
        ],
    )
    def sc_kernel(h_hbm, src_hbm, dst_hbm, n0_hbm, n1_hbm, n2_hbm,
                  out_hbm, idx5, rows, sems):
        id_hbm = [src_hbm, dst_hbm, n0_hbm, n1_hbm, n2_hbm]
        wid = lax.axis_index("s") * NC + lax.axis_index("c")
        wbase = wid * epw
        gsem = sems[:nbuf]
        wsem = sems[nbuf:2 * nbuf]

        idc = [pltpu.async_copy(id_hbm[k].at[pl.ds(wbase, epw)], idx5[k],
                                sems[-1])
               for k in range(5)]
        for c in idc:
            c.wait()

        def idx_slice(k, off):
            return idx5[k].at[pl.ds(off, CHUNK)]

        def g_issue(ci, bslot):
            off = ci * CHUNK
            for k in range(5):
                pltpu.async_copy(h_hbm.at[idx_slice(k, off)],
                                 rows[bslot][k], gsem[bslot])

        def g_drain(bslot):
            for k in range(5):
                pltpu.make_async_copy(h_hbm.at[pl.ds(0, CHUNK)],
                                      rows[bslot][k], gsem[bslot]).wait()

        for bslot in range(nbuf):
            g_issue(bslot, bslot)

        def pair_body(ci, carry):
            for bslot in range(nbuf):
                chunk = ci + bslot
                base = wbase + chunk * CHUNK
                g_drain(bslot)
                wcp = [pltpu.async_copy(rows[bslot][k],
                                        out_hbm.at[k, pl.ds(base, CHUNK)],
                                        wsem[bslot])
                       for k in range(5)]
                for c in wcp:
                    c.wait()
                g_issue(lax.rem(chunk + nbuf, nchunks), bslot)
            return carry

        lax.fori_loop(0, nchunks // nbuf, lambda i, c: pair_body(i * nbuf, c),
                      0)
        for bslot in range(nbuf):
            g_drain(bslot)

    return sc_kernel(h, src, dst, n0, n1, n2)


def _dot_loss_body(g_ref, out_ref):
    u = lax.bitcast_convert_type(g_ref[...], jnp.uint32)
    flo = lax.bitcast_convert_type(u << jnp.uint32(16), jnp.float32)
    fhi = lax.bitcast_convert_type(u & jnp.uint32(0xFFFF0000), jnp.float32)

    def dot(k):
        return jnp.sum(flo[0] * flo[k] + fhi[0] * fhi[k], axis=-1)

    pos, n0, n1, n2 = dot(1), dot(2), dot(3), dot(4)

    def sp(x):
        return jnp.maximum(x, 0.0) + jnp.log1p(jnp.exp(-jnp.abs(x)))

    out_ref[...] = (sp(-pos)) + 10.0 * (sp(n0) + sp(n1) + sp(n2))


def _dot_loss(g):
    b = g.shape[1]
    nb = 2048
    return pl.pallas_call(
        _dot_loss_body,
        grid=(b // nb,),
        in_specs=[pl.BlockSpec((5, nb, D_WORDS), lambda i: (0, i, 0))],
        out_specs=pl.BlockSpec((nb,), lambda i: (i,)),
        out_shape=jax.ShapeDtypeStruct((b,), jnp.float32),
    )(g)


def kernel(nf, W, src, dst, neg):
    b = src.shape[0]
    h = _project(nf, W)
    ridx = jax.random.randint(jax.random.key(42), (b, 3), 0, b)
    r0, r1, r2 = (ridx[:, k].astype(jnp.int32) for k in range(3))
    src32, dst32 = src.astype(jnp.int32), dst.astype(jnp.int32)
    neg32 = neg.astype(jnp.int32)
    n0, n1, n2 = _sc_negids(neg32, r0, r1, r2)
    nch = 4 if b % (4 * NW * CHUNK * 2) == 0 else 1
    cb = b // nch
    outs = []
    for i in range(nch):
        lo, hi = i * cb, (i + 1) * cb
        g = _sc_gather(h, src32[lo:hi], dst32[lo:hi],
                       n0[lo:hi], n1[lo:hi], n2[lo:hi])
        outs.append(_dot_loss(g))
    return jnp.concatenate(outs) if nch > 1 else outs[0]


# speedup vs baseline: 16.9279x; 1.0236x over previous
import functools

import jax
import jax.numpy as jnp
from jax import lax
from jax.experimental import pallas as pl
from jax.experimental.pallas import tpu as pltpu
from jax.experimental.pallas import tpu_sc as plsc

D_MODEL = 256
D_WORDS = D_MODEL // 2
NC, NS = 2, 16
NW = NC * NS
CHUNK = 32


def _rne_bf16_hi(x):
    u = lax.bitcast_convert_type(x, jnp.uint32)
    r = u + jnp.uint32(0x7FFF) + ((u >> jnp.uint32(16)) & jnp.uint32(1))
    return r & jnp.uint32(0xFFFF0000)


def _matmul_body(nf_ref, w_ref, out_ref):
    acc = jnp.dot(nf_ref[...], w_ref[...],
                  preferred_element_type=jnp.float32)
    lo = _rne_bf16_hi(acc[:, :D_WORDS]) >> jnp.uint32(16)
    hi = _rne_bf16_hi(acc[:, D_WORDS:])
    out_ref[...] = lax.bitcast_convert_type(hi | lo, jnp.int32)


def _project(nf, W):
    n, d = nf.shape
    bm = 2000 if n % 2000 == 0 else 512
    return pl.pallas_call(
        _matmul_body,
        grid=(pl.cdiv(n, bm),),
        in_specs=[pl.BlockSpec((bm, d), lambda i: (i, 0)),
                  pl.BlockSpec((d, d), lambda i: (0, 0))],
        out_specs=pl.BlockSpec((bm, D_WORDS), lambda i: (i, 0)),
        out_shape=jax.ShapeDtypeStruct((n, D_WORDS), jnp.int32),
    )(nf, W)


def _sc_negids(neg, r0, r1, r2):
    b = r0.shape[0]
    epw = b // NW
    mesh = plsc.VectorSubcoreMesh(core_axis_name="c", subcore_axis_name="s",
                                  num_cores=NC, num_subcores=NS)

    @functools.partial(
        pl.kernel,
        out_type=[jax.ShapeDtypeStruct((b,), jnp.int32) for _ in range(3)],
        mesh=mesh,
        compiler_params=pltpu.CompilerParams(needs_layout_passes=False),
        scratch_types=[
            [pltpu.VMEM((epw,), jnp.int32) for _ in range(3)],
            [pltpu.VMEM((epw,), jnp.int32) for _ in range(3)],
            pltpu.SemaphoreType.DMA,
        ],
    )
    def negid_kernel(neg_hbm, r0_hbm, r1_hbm, r2_hbm,
                     o0_hbm, o1_hbm, o2_hbm, rall, negid, sem):
        ridx_hbm = [r0_hbm, r1_hbm, r2_hbm]
        out_hbm = [o0_hbm, o1_hbm, o2_hbm]
        wid = lax.axis_index("s") * NC + lax.axis_index("c")
        wbase = wid * epw
        for k in range(3):
            pltpu.sync_copy(ridx_hbm[k].at[pl.ds(wbase, epw)], rall[k])
        gcp = [pltpu.async_copy(neg_hbm.at[rall[k]], negid[k], sem)
               for k in range(3)]
        for c in gcp:
            c.wait()
        wcp = [pltpu.async_copy(negid[k], out_hbm[k].at[pl.ds(wbase, epw)],
                                sem)
               for k in range(3)]
        for c in wcp:
            c.wait()

    return negid_kernel(neg, r0, r1, r2)


def _sc_gather(h, src, dst, n0, n1, n2):
    b = src.shape[0]
    epw = b // NW
    nchunks = epw // CHUNK
    nbuf = 2
    mesh = plsc.VectorSubcoreMesh(core_axis_name="c", subcore_axis_name="s",
                                  num_cores=NC, num_subcores=NS)

    @functools.partial(
        pl.kernel,
        out_type=jax.ShapeDtypeStruct((5, b, D_WORDS), jnp.int32),
        mesh=mesh,
        compiler_params=pltpu.CompilerParams(needs_layout_passes=False),
        scratch_types=[
            [pltpu.VMEM((epw,), jnp.int32) for _ in range(5)],
            [[pltpu.VMEM((CHUNK, D_WORDS), jnp.int32) for _ in range(5)]
             for _ in range(nbuf)],
            [pltpu.SemaphoreType.DMA for _ in range(2 * nbuf + 1)],
        ],
    )
    def sc_kernel(h_hbm, src_hbm, dst_hbm, n0_hbm, n1_hbm, n2_hbm,
                  out_hbm, idx5, rows, sems):
        id_hbm = [src_hbm, dst_hbm, n0_hbm, n1_hbm, n2_hbm]
        wid = lax.axis_index("s") * NC + lax.axis_index("c")
        wbase = wid * epw
        gsem = sems[:nbuf]
        wsem = sems[nbuf:2 * nbuf]

        idc = [pltpu.async_copy(id_hbm[k].at[pl.ds(wbase, epw)], idx5[k],
                                sems[-1])
               for k in range(5)]
        for c in idc:
            c.wait()

        def idx_slice(k, off):
            return idx5[k].at[pl.ds(off, CHUNK)]

        def g_issue(ci, bslot):
            off = ci * CHUNK
            for k in range(5):
                pltpu.async_copy(h_hbm.at[idx_slice(k, off)],
                                 rows[bslot][k], gsem[bslot])

        def g_drain(bslot):
            for k in range(5):
                pltpu.make_async_copy(h_hbm.at[pl.ds(0, CHUNK)],
                                      rows[bslot][k], gsem[bslot]).wait()

        for bslot in range(nbuf):
            g_issue(bslot, bslot)

        def pair_body(ci, carry):
            for bslot in range(nbuf):
                chunk = ci + bslot
                base = wbase + chunk * CHUNK
                g_drain(bslot)
                wcp = [pltpu.async_copy(rows[bslot][k],
                                        out_hbm.at[k, pl.ds(base, CHUNK)],
                                        wsem[bslot])
                       for k in range(5)]
                for c in wcp:
                    c.wait()
                g_issue(lax.rem(chunk + nbuf, nchunks), bslot)
            return carry

        lax.fori_loop(0, nchunks // nbuf, lambda i, c: pair_body(i * nbuf, c),
                      0)
        for bslot in range(nbuf):
            g_drain(bslot)

    return sc_kernel(h, src, dst, n0, n1, n2)


def _dot_loss_body(g_ref, out_ref):
    u = lax.bitcast_convert_type(g_ref[...], jnp.uint32)
    flo = lax.bitcast_convert_type(u << jnp.uint32(16), jnp.float32)
    fhi = lax.bitcast_convert_type(u & jnp.uint32(0xFFFF0000), jnp.float32)

    def dot(k):
        return jnp.sum(flo[0] * flo[k] + fhi[0] * fhi[k], axis=-1)

    pos, n0, n1, n2 = dot(1), dot(2), dot(3), dot(4)

    def sp(x):
        return jnp.maximum(x, 0.0) + jnp.log1p(jnp.exp(-jnp.abs(x)))

    out_ref[...] = (sp(-pos)) + 10.0 * (sp(n0) + sp(n1) + sp(n2))


def _dot_loss(g):
    b = g.shape[1]
    rows, nbr = b // 128, 16
    g4 = g.reshape(5, rows, 128, D_WORDS)
    out2 = pl.pallas_call(
        _dot_loss_body,
        grid=(rows // nbr,),
        in_specs=[pl.BlockSpec((5, nbr, 128, D_WORDS),
                               lambda i: (0, i, 0, 0))],
        out_specs=pl.BlockSpec((nbr, 128), lambda i: (i, 0)),
        out_shape=jax.ShapeDtypeStruct((rows, 128), jnp.float32),
    )(g4)
    return out2.reshape(b)


def kernel(nf, W, src, dst, neg):
    b = src.shape[0]
    h = _project(nf, W)
    ridx = jax.random.randint(jax.random.key(42), (b, 3), 0, b)
    r0, r1, r2 = (ridx[:, k].astype(jnp.int32) for k in range(3))
    src32, dst32 = src.astype(jnp.int32), dst.astype(jnp.int32)
    neg32 = neg.astype(jnp.int32)
    n0, n1, n2 = _sc_negids(neg32, r0, r1, r2)
    nch = 4 if b % (4 * NW * CHUNK * 2) == 0 else 1
    cb = b // nch
    outs = []
    for i in range(nch):
        lo, hi = i * cb, (i + 1) * cb
        g = _sc_gather(h, src32[lo:hi], dst32[lo:hi],
                       n0[lo:hi], n1[lo:hi], n2[lo:hi])
        outs.append(_dot_loss(g))
    return jnp.concatenate(outs) if nch > 1 else outs[0]
